# baseline (speedup 1.0000x reference)
import jax
import jax.numpy as jnp
from jax import lax
from jax.experimental import pallas as pl
from jax.experimental.pallas import tpu as pltpu
from jax.experimental.pallas import tpu_sc as plsc

VOCAB = 100000
EMBED = 128
NC = 2
NS = 16
NW = NC * NS
CHUNK = 128
NBUF = 6
NPAIR = NBUF // 2
SC_CPW = 42
RPS = 2048


def _sc_body(idx_hbm, table_hbm, out_hbm, idx_v, rows_v, *sems):
    nchunk = idx_hbm.shape[1]
    nslot = -(-nchunk // NBUF) * NBUF
    gsems = sems[:NBUF]
    ssems = sems[NBUF:]
    wid = lax.axis_index("s") * NC + lax.axis_index("c")
    base = wid * nchunk

    pltpu.sync_copy(idx_hbm.at[wid], idx_v)

    def gather_start(c, b):
        pltpu.async_copy(table_hbm.at[idx_v.at[c]], rows_v.at[b], gsems[b])

    def gather_wait(c, b):
        pltpu.make_async_copy(
            table_hbm.at[idx_v.at[c]], rows_v.at[b], gsems[b]).wait()

    def scatter_start(c, p):
        pltpu.async_copy(rows_v.at[pl.ds(2 * p, 2)],
                         out_hbm.at[pl.ds(base + c, 2)], ssems[p])

    def scatter_wait(c, p):
        pltpu.make_async_copy(rows_v.at[pl.ds(2 * p, 2)],
                              out_hbm.at[pl.ds(base + c, 2)], ssems[p]).wait()

    for b in range(NBUF):
        if b < nchunk:
            gather_start(b, b)

    @pl.loop(0, nslot, step=NBUF)
    def _round(g):
        for p in range(NPAIR):
            c = g + 2 * p

            @pl.when(c < nchunk)
            def _():
                gather_wait(c, 2 * p)
                gather_wait(c + 1, 2 * p + 1)
                scatter_start(c, p)
        for p in range(NPAIR):
            c = g + 2 * p

            @pl.when(c < nchunk)
            def _():
                scatter_wait(c, p)

            nxt = g + NBUF + 2 * p

            @pl.when(nxt < nchunk)
            def _():
                gather_start(nxt, 2 * p)
                gather_start(nxt + 1, 2 * p + 1)


def _tc_body(idx_ref, table_ref, out_ref):
    @pl.loop(0, RPS, unroll=8)
    def _row(i):
        out_ref[i, :] = table_ref[idx_ref[i], :]


def kernel(x, embed):
    B, T = x.shape
    n = B * T
    n_sc = NW * SC_CPW * CHUNK
    n_tc = n - n_sc
    nstep = n_tc // RPS
    idx = x.reshape(n).astype(jnp.int32)
    idx_sc = idx[:n_sc].reshape(NW, SC_CPW, CHUNK)
    idx_tc = idx[n_sc:]

    sc_call = pl.kernel(
        _sc_body,
        out_type=jax.ShapeDtypeStruct((n // CHUNK, CHUNK, EMBED),
                                      jnp.float32),
        mesh=plsc.VectorSubcoreMesh(core_axis_name="c", subcore_axis_name="s"),
        scratch_types=[
            pltpu.VMEM((SC_CPW, CHUNK), jnp.int32),
            pltpu.VMEM((NBUF, CHUNK, EMBED), jnp.float32),
        ] + [pltpu.SemaphoreType.DMA] * (NBUF + NPAIR),
    )
    out_sc = sc_call(idx_sc, embed)

    out_tc = pl.pallas_call(
        _tc_body,
        out_shape=jax.ShapeDtypeStruct((n_tc, EMBED), jnp.float32),
        grid=(nstep,),
        in_specs=[
            pl.BlockSpec((RPS,), lambda i: (i,), memory_space=pltpu.SMEM),
            pl.BlockSpec((VOCAB, EMBED), lambda i: (0, 0)),
        ],
        out_specs=pl.BlockSpec((RPS, EMBED), lambda i: (i, 0)),
        compiler_params=pltpu.CompilerParams(
            vmem_limit_bytes=100 * 1024 * 1024),
    )(idx_tc, embed)

    out_tc3 = out_tc.reshape(n_tc // CHUNK, CHUNK, EMBED)
    out = lax.dynamic_update_slice(out_sc, out_tc3, (n_sc // CHUNK, 0, 0))
    return out.reshape(B, T, EMBED)


# speedup vs baseline: 1.2948x; 1.2948x over previous
import jax
import jax.numpy as jnp
from jax import lax
from jax.experimental import pallas as pl
from jax.experimental.pallas import tpu as pltpu
from jax.experimental.pallas import tpu_sc as plsc

VOCAB = 100000
EMBED = 128
NC = 2
NS = 16
NW = NC * NS
CHUNK = 128
NBUF = 6
NPAIR = NBUF // 2


def _gather_body(idx_hbm, table_hbm, out_hbm, idx_v, rows_v, *sems):
    nchunk = idx_hbm.shape[1]
    nslot = -(-nchunk // NBUF) * NBUF
    gsems = sems[:NBUF]
    ssems = sems[NBUF:]
    wid = lax.axis_index("s") * NC + lax.axis_index("c")
    base = wid * nchunk

    pltpu.sync_copy(idx_hbm.at[wid], idx_v)

    def gather_start(c, b):
        pltpu.async_copy(table_hbm.at[idx_v.at[c]], rows_v.at[b], gsems[b])

    def gather_wait(c, b):
        pltpu.make_async_copy(
            table_hbm.at[idx_v.at[c]], rows_v.at[b], gsems[b]).wait()

    def scatter_start(c, p):
        pltpu.async_copy(rows_v.at[pl.ds(2 * p, 2)],
                         out_hbm.at[pl.ds(base + c, 2)], ssems[p])

    def scatter_wait(c, p):
        pltpu.make_async_copy(rows_v.at[pl.ds(2 * p, 2)],
                              out_hbm.at[pl.ds(base + c, 2)], ssems[p]).wait()

    for b in range(NBUF):
        if b < nchunk:
            gather_start(b, b)

    @pl.loop(0, nslot, step=NBUF)
    def _round(g):
        for p in range(NPAIR):
            c = g + 2 * p

            @pl.when(c < nchunk)
            def _():
                gather_wait(c, 2 * p)
                gather_wait(c + 1, 2 * p + 1)
                scatter_start(c, p)
        for p in range(NPAIR):
            c = g + 2 * p

            @pl.when(c < nchunk)
            def _():
                scatter_wait(c, p)

            nxt = g + NBUF + 2 * p

            @pl.when(nxt < nchunk)
            def _():
                gather_start(nxt, 2 * p)
                gather_start(nxt + 1, 2 * p + 1)


def kernel(x, embed):
    B, T = x.shape
    n = B * T
    per_w = n // NW
    nchunk = per_w // CHUNK
    idx3 = x.reshape(NW, nchunk, CHUNK).astype(jnp.int32)

    call = pl.kernel(
        _gather_body,
        out_type=jax.ShapeDtypeStruct((n // CHUNK, CHUNK, EMBED), jnp.float32),
        mesh=plsc.VectorSubcoreMesh(core_axis_name="c", subcore_axis_name="s"),
        scratch_types=[
            pltpu.VMEM((nchunk, CHUNK), jnp.int32),
            pltpu.VMEM((NBUF, CHUNK, EMBED), jnp.float32),
        ] + [pltpu.SemaphoreType.DMA] * (NBUF + NPAIR),
    )
    out = call(idx3, embed)
    return out.reshape(B, T, EMBED)
